# speedup vs baseline: 1.6327x; 1.6327x over previous
import functools

import numpy as np
import jax
import jax.numpy as jnp
from jax.experimental import pallas as pl
from jax.experimental.pallas import tpu as pltpu

_NBINS = 30
_VMIN = -6.0
_VMAX = 6.0
_BW = abs((_VMAX - _VMIN) / (_NBINS - 2))

_S64 = []
_start = _VMIN
for _ in range(_NBINS - 1):
    _S64.append(_start)
    _start = _start + _BW
_S32 = np.asarray(_S64, dtype=np.float32)


def _hist_body(x_ref, o_ref, acc_ref, *, n_j):
    j = pl.program_id(1)

    @pl.when(j == 0)
    def _init():
        acc_ref[...] = jnp.zeros_like(acc_ref)

    x = x_ref[...]
    parts = [jnp.sum((x < np.float32(_VMIN)).astype(jnp.float32), axis=1)]
    for k in range(29):
        parts.append(jnp.sum((x >= _S32[k]).astype(jnp.float32), axis=1))
    acc_ref[...] += jnp.stack(parts, axis=1)

    @pl.when(j == n_j - 1)
    def _finalize():
        a = acc_ref[...]
        cols = [a[:, 0], a[:, 29]]
        for i in range(28):
            cols.append(a[:, 1 + i] - a[:, 2 + i])
        o_ref[...] = jnp.stack(cols, axis=1)


def kernel(x):
    n, m = x.shape
    br = min(256, n)
    bc = min(4096, m)
    n_i, n_j = n // br, m // bc
    out = pl.pallas_call(
        functools.partial(_hist_body, n_j=n_j),
        grid=(n_i, n_j),
        in_specs=[pl.BlockSpec((br, bc), lambda i, j: (i, j))],
        out_specs=pl.BlockSpec((br, _NBINS), lambda i, j: (i, 0)),
        out_shape=jax.ShapeDtypeStruct((n, _NBINS), jnp.float32),
        scratch_shapes=[pltpu.VMEM((br, _NBINS), jnp.float32)],
        compiler_params=pltpu.CompilerParams(
            dimension_semantics=("parallel", "arbitrary"),
        ),
    )(x)
    return out
